# baseline (speedup 1.0000x reference)
import numpy as np
import jax
import jax.numpy as jnp
from jax.experimental import pallas as pl
from jax.experimental.pallas import tpu as pltpu
from jax.experimental.pallas import tpu_sc as plsc

_B = 16384
_F = 26
_D = 64
_NF = _F + 1
_V = 2600000

_SPLIT = 1310720
_PACK_C = 8192
_N_IN_BLOCKS = (_V + _PACK_C - 1) // _PACK_C - 1

_GATHER_WINDOW = 128


def _pack_kernel(x0_ref, x1_ref, o_ref):
    x = jnp.concatenate([x0_ref[...], x1_ref[...]], axis=0)
    o_ref[...] = jnp.transpose(x)


def _pack(wt):
    nj = _SPLIT // _PACK_C
    return pl.pallas_call(
        _pack_kernel,
        grid=(nj,),
        in_specs=[
            pl.BlockSpec((_D, _PACK_C), lambda j: (0, j)),
            pl.BlockSpec((_D, _PACK_C),
                         lambda j: (0, jnp.minimum(j + nj, _N_IN_BLOCKS))),
        ],
        out_specs=pl.BlockSpec((_PACK_C, 128), lambda j: (j, 0)),
        out_shape=jax.ShapeDtypeStruct((_SPLIT, 128), jnp.float32),
        compiler_params=pltpu.CompilerParams(
            dimension_semantics=("parallel",)),
    )(wt, wt)


def _sc_gather(table, flat_idx):
    n = flat_idx.shape[0]
    d = table.shape[1]
    idx2 = flat_idx.reshape(1, n)

    @pl.kernel(
        out_type=jax.ShapeDtypeStruct((n, d), table.dtype),
        mesh=plsc.VectorSubcoreMesh(core_axis_name="core",
                                    subcore_axis_name="subcore"),
    )
    def gather_kernel(x_hbm, i_hbm, o_hbm):
        def body(i_vmem, o_vmem):
            pltpu.sync_copy(x_hbm.at[i_vmem.at[0]], o_vmem)

        pltpu.emit_pipeline(
            body,
            grid=(n // _GATHER_WINDOW,),
            in_specs=[pl.BlockSpec((1, _GATHER_WINDOW),
                                   index_map=lambda i: (0, i))],
            out_specs=[pl.BlockSpec((_GATHER_WINDOW, d),
                                    index_map=lambda i: (i, 0))],
            core_axis_name=("core", "subcore"),
            dimension_semantics=(pltpu.PARALLEL,),
        )(i_hbm, o_hbm)

    return gather_kernel(table, idx2)


def _dense_mlp_kernel(x_ref, w1_ref, b1_ref, w2_ref, b2_ref, w3_ref, b3_ref,
                      o_ref):
    x = x_ref[...]
    h = jnp.maximum(x @ w1_ref[...] + b1_ref[...], 0.0)
    h = jnp.maximum(h @ w2_ref[...] + b2_ref[...], 0.0)
    h = jnp.maximum(h @ w3_ref[...] + b3_ref[...], 0.0)
    o_ref[...] = h


def _dense_mlp(x_pad, w1p, b1, w2, b2, w3, b3):
    bb = 2048
    full = lambda a: pl.BlockSpec(a.shape, lambda i: (0,) * a.ndim)
    return pl.pallas_call(
        _dense_mlp_kernel,
        grid=(_B // bb,),
        in_specs=[pl.BlockSpec((bb, x_pad.shape[1]), lambda i: (i, 0)),
                  full(w1p), full(b1), full(w2), full(b2), full(w3), full(b3)],
        out_specs=pl.BlockSpec((bb, _D), lambda i: (i, 0)),
        out_shape=jax.ShapeDtypeStruct((_B, _D), jnp.float32),
    )(x_pad, w1p, b1, w2, b2, w3, b3)


def _main_kernel(wide_ref, h_ref, d_ref, w1_ref, b1_ref, w2_ref, b2_ref,
                 w3_ref, b3_ref, w4_ref, b4_ref, w5_ref, b5_ref, o_ref):
    wide = wide_ref[...]
    hsel = h_ref[...]
    lo = wide[:, :_D]
    hi = wide[:, _D:]
    e2 = jnp.where(hsel[:, None] > 0.5, hi, lo)
    bb = e2.shape[0] // _F
    d = d_ref[...]
    c = jnp.concatenate([d[:, None, :], e2.reshape(bb, _F, _D)],
                        axis=1)
    inter = jax.lax.dot_general(
        c, c, dimension_numbers=(((2,), (2,)), ((0,), (0,))),
        preferred_element_type=jnp.float32)
    parts = [d]
    for i in range(1, _NF):
        parts.append(inter[:, i, :i])
    parts.append(jnp.zeros((bb, 1), jnp.float32))
    x = jnp.concatenate(parts, axis=1)
    x = jnp.maximum(x @ w1_ref[...] + b1_ref[...], 0.0)
    x = jnp.maximum(x @ w2_ref[...] + b2_ref[...], 0.0)
    x = jnp.maximum(x @ w3_ref[...] + b3_ref[...], 0.0)
    x = jnp.maximum(x @ w4_ref[...] + b4_ref[...], 0.0)
    o_ref[...] = x @ w5_ref[...] + b5_ref[...]


def _main(wide, hsel, d, w1p, b1, w2, b2, w3, b3, w4, b4, w5, b5):
    bb = 512
    full = lambda a: pl.BlockSpec(a.shape, lambda i: (0,) * a.ndim)
    return pl.pallas_call(
        _main_kernel,
        grid=(_B // bb,),
        in_specs=[pl.BlockSpec((bb * _F, 128), lambda i: (i, 0)),
                  pl.BlockSpec((bb * _F,), lambda i: (i,)),
                  pl.BlockSpec((bb, _D), lambda i: (i, 0)),
                  full(w1p), full(b1), full(w2), full(b2), full(w3), full(b3),
                  full(w4), full(b4), full(w5), full(b5)],
        out_specs=pl.BlockSpec((bb, 1), lambda i: (i, 0)),
        out_shape=jax.ShapeDtypeStruct((_B, 1), jnp.float32),
    )(wide, hsel, d, w1p, b1, w2, b2, w3, b3, w4, b4, w5, b5)


def kernel(dense_features, sparse_indices, offsets, W_embed, dense_params,
           over_params):
    flat_idx = (sparse_indices + offsets[None, :]).reshape(-1).astype(jnp.int32)
    h = flat_idx >= _SPLIT
    q = jnp.where(h, flat_idx - _SPLIT, flat_idx).astype(jnp.int32)

    (w1d, b1d), (w2d, b2d), (w3d, b3d) = dense_params
    x_pad = jnp.pad(dense_features, ((0, 0), (0, 16 - dense_features.shape[1])))
    w1d_pad = jnp.pad(w1d, ((0, 16 - w1d.shape[0]), (0, 0)))

    (w1o, b1o), (w2o, b2o), (w3o, b3o), (w4o, b4o), (w5o, b5o) = over_params
    w1o_pad = jnp.pad(w1o, ((0, 416 - w1o.shape[0]), (0, 0)))

    r2 = lambda b: b.reshape(1, -1)

    wp = _pack(W_embed.T)

    wide = _sc_gather(wp, q)

    d = _dense_mlp(x_pad, w1d_pad, r2(b1d), w2d, r2(b2d), w3d, r2(b3d))

    hsel = h.astype(jnp.float32)
    logits = _main(wide, hsel, d, w1o_pad, r2(b1o), w2o, r2(b2o), w3o, r2(b3o),
                   w4o, r2(b4o), w5o, r2(b5o))
    return logits


# speedup vs baseline: 1.0973x; 1.0973x over previous
import numpy as np
import jax
import jax.numpy as jnp
from jax.experimental import pallas as pl
from jax.experimental.pallas import tpu as pltpu
from jax.experimental.pallas import tpu_sc as plsc

_B = 16384
_F = 26
_D = 64
_NF = _F + 1
_V = 2600000

_SPLIT = 1310720
_PACK_C = 8192
_N_IN_BLOCKS = (_V + _PACK_C - 1) // _PACK_C - 1

_GATHER_WINDOW = 128


def _pack_kernel(x0_ref, x1_ref, o_ref):
    x = jnp.concatenate([x0_ref[...], x1_ref[...]], axis=0)
    o_ref[...] = jnp.transpose(x)


def _pack(wt):
    nj = _SPLIT // _PACK_C
    return pl.pallas_call(
        _pack_kernel,
        grid=(nj,),
        in_specs=[
            pl.BlockSpec((_D, _PACK_C), lambda j: (0, j)),
            pl.BlockSpec((_D, _PACK_C),
                         lambda j: (0, jnp.minimum(j + nj, _N_IN_BLOCKS))),
        ],
        out_specs=pl.BlockSpec((_PACK_C, 128), lambda j: (j, 0)),
        out_shape=jax.ShapeDtypeStruct((_SPLIT, 128), jnp.float32),
        compiler_params=pltpu.CompilerParams(
            dimension_semantics=("parallel",)),
    )(wt, wt)


def _sc_gather(table, flat_idx):
    n = flat_idx.shape[0]
    d = table.shape[1]
    idx2 = flat_idx.reshape(1, n)

    @pl.kernel(
        out_type=jax.ShapeDtypeStruct((n, d), table.dtype),
        mesh=plsc.VectorSubcoreMesh(core_axis_name="core",
                                    subcore_axis_name="subcore"),
    )
    def gather_kernel(x_hbm, i_hbm, o_hbm):
        def body(i_vmem, o_vmem):
            pltpu.sync_copy(x_hbm.at[i_vmem.at[0]], o_vmem)

        pltpu.emit_pipeline(
            body,
            grid=(n // _GATHER_WINDOW,),
            in_specs=[pl.BlockSpec((1, _GATHER_WINDOW),
                                   index_map=lambda i: (0, i))],
            out_specs=[pl.BlockSpec((_GATHER_WINDOW, d),
                                    index_map=lambda i: (i, 0))],
            core_axis_name=("core", "subcore"),
            dimension_semantics=(pltpu.PARALLEL,),
        )(i_hbm, o_hbm)

    return gather_kernel(table, idx2)


def _dense_mlp_kernel(x_ref, w1_ref, b1_ref, w2_ref, b2_ref, w3_ref, b3_ref,
                      o_ref):
    x = x_ref[...]
    h = jnp.maximum(x @ w1_ref[...] + b1_ref[...], 0.0)
    h = jnp.maximum(h @ w2_ref[...] + b2_ref[...], 0.0)
    h = jnp.maximum(h @ w3_ref[...] + b3_ref[...], 0.0)
    o_ref[...] = h


def _dense_mlp(x_pad, w1p, b1, w2, b2, w3, b3):
    bb = 2048
    full = lambda a: pl.BlockSpec(a.shape, lambda i: (0,) * a.ndim)
    return pl.pallas_call(
        _dense_mlp_kernel,
        grid=(_B // bb,),
        in_specs=[pl.BlockSpec((bb, x_pad.shape[1]), lambda i: (i, 0)),
                  full(w1p), full(b1), full(w2), full(b2), full(w3), full(b3)],
        out_specs=pl.BlockSpec((bb, _D), lambda i: (i, 0)),
        out_shape=jax.ShapeDtypeStruct((_B, _D), jnp.float32),
    )(x_pad, w1p, b1, w2, b2, w3, b3)


def _main_kernel(wide_ref, h_ref, d_ref, w1_ref, b1_ref, w2_ref, b2_ref,
                 w3_ref, b3_ref, w4_ref, b4_ref, w5_ref, b5_ref, o_ref):
    wide = wide_ref[...]
    hsel = h_ref[...]
    lo = wide[:, :_D]
    hi = wide[:, _D:]
    e2 = jnp.where(hsel[:, None] > 0.5, hi, lo)
    bb = e2.shape[0] // _F
    d = d_ref[...]
    c = jnp.concatenate([d[:, None, :], e2.reshape(bb, _F, _D)],
                        axis=1)
    inter = jax.lax.dot_general(
        c, c, dimension_numbers=(((2,), (2,)), ((0,), (0,))),
        preferred_element_type=jnp.float32)
    parts = [d]
    for i in range(1, _NF):
        parts.append(inter[:, i, :i])
    parts.append(jnp.zeros((bb, 1), jnp.float32))
    x = jnp.concatenate(parts, axis=1)
    x = jnp.maximum(x @ w1_ref[...] + b1_ref[...], 0.0)
    x = jnp.maximum(x @ w2_ref[...] + b2_ref[...], 0.0)
    x = jnp.maximum(x @ w3_ref[...] + b3_ref[...], 0.0)
    x = jnp.maximum(x @ w4_ref[...] + b4_ref[...], 0.0)
    o_ref[...] = x @ w5_ref[...] + b5_ref[...]


def _main(wide, hsel, d, w1p, b1, w2, b2, w3, b3, w4, b4, w5, b5):
    bb = 512
    n = wide.shape[0] // _F
    full = lambda a: pl.BlockSpec(a.shape, lambda i: (0,) * a.ndim)
    return pl.pallas_call(
        _main_kernel,
        grid=(n // bb,),
        in_specs=[pl.BlockSpec((bb * _F, 128), lambda i: (i, 0)),
                  pl.BlockSpec((bb * _F,), lambda i: (i,)),
                  pl.BlockSpec((bb, _D), lambda i: (i, 0)),
                  full(w1p), full(b1), full(w2), full(b2), full(w3), full(b3),
                  full(w4), full(b4), full(w5), full(b5)],
        out_specs=pl.BlockSpec((bb, 1), lambda i: (i, 0)),
        out_shape=jax.ShapeDtypeStruct((n, 1), jnp.float32),
    )(wide, hsel, d, w1p, b1, w2, b2, w3, b3, w4, b4, w5, b5)


def kernel(dense_features, sparse_indices, offsets, W_embed, dense_params,
           over_params):
    flat_idx = (sparse_indices + offsets[None, :]).reshape(-1).astype(jnp.int32)
    h = flat_idx >= _SPLIT
    q = jnp.where(h, flat_idx - _SPLIT, flat_idx).astype(jnp.int32)

    (w1d, b1d), (w2d, b2d), (w3d, b3d) = dense_params
    x_pad = jnp.pad(dense_features, ((0, 0), (0, 16 - dense_features.shape[1])))
    w1d_pad = jnp.pad(w1d, ((0, 16 - w1d.shape[0]), (0, 0)))

    (w1o, b1o), (w2o, b2o), (w3o, b3o), (w4o, b4o), (w5o, b5o) = over_params
    w1o_pad = jnp.pad(w1o, ((0, 416 - w1o.shape[0]), (0, 0)))

    r2 = lambda b: b.reshape(1, -1)

    wp = _pack(W_embed.T)

    half = (_B * _F) // 2
    wide1 = _sc_gather(wp, q[:half])
    wide2 = _sc_gather(wp, q[half:])

    d = _dense_mlp(x_pad, w1d_pad, r2(b1d), w2d, r2(b2d), w3d, r2(b3d))

    hsel = h.astype(jnp.float32)
    ow = (w1o_pad, r2(b1o), w2o, r2(b2o), w3o, r2(b3o), w4o, r2(b4o), w5o,
          r2(b5o))
    logits1 = _main(wide1, hsel[:half], d[:_B // 2], *ow)
    logits2 = _main(wide2, hsel[half:], d[_B // 2:], *ow)
    return jnp.concatenate([logits1, logits2], axis=0)


# speedup vs baseline: 1.1859x; 1.0808x over previous
import numpy as np
import jax
import jax.numpy as jnp
from jax.experimental import pallas as pl
from jax.experimental.pallas import tpu as pltpu
from jax.experimental.pallas import tpu_sc as plsc

_B = 16384
_F = 26
_D = 64
_NF = _F + 1
_V = 2600000

_SPLIT4 = 655360
_PACK_C = 8192
_N_IN_BLOCKS = (_V + _PACK_C - 1) // _PACK_C - 1

_GATHER_WINDOW = 128


def _pack_kernel(x0_ref, x1_ref, x2_ref, x3_ref, o_ref):
    t = jnp.transpose(jnp.concatenate([x0_ref[...], x2_ref[...]],
                                      axis=0))
    u = jnp.transpose(jnp.concatenate([x1_ref[...], x3_ref[...]],
                                      axis=0))
    tb = jax.lax.bitcast_convert_type(t, jnp.int32)
    ub = jax.lax.bitcast_convert_type(u, jnp.int32)
    o_ref[...] = jnp.bitwise_or(
        jax.lax.shift_right_logical(tb, 16),
        jnp.bitwise_and(ub, jnp.int32(-65536)))


def _pack(wt):
    nj = _SPLIT4 // _PACK_C

    def imap(k):
        return lambda j: (0, jnp.minimum(j + k * nj, _N_IN_BLOCKS))

    return pl.pallas_call(
        _pack_kernel,
        grid=(nj,),
        in_specs=[pl.BlockSpec((_D, _PACK_C), imap(0)),
                  pl.BlockSpec((_D, _PACK_C), imap(1)),
                  pl.BlockSpec((_D, _PACK_C), imap(2)),
                  pl.BlockSpec((_D, _PACK_C), imap(3))],
        out_specs=pl.BlockSpec((_PACK_C, 128), lambda j: (j, 0)),
        out_shape=jax.ShapeDtypeStruct((_SPLIT4, 128), jnp.int32),
    )(wt, wt, wt, wt)


def _sc_gather(table, flat_idx):
    n = flat_idx.shape[0]
    d = table.shape[1]
    idx2 = flat_idx.reshape(1, n)

    @pl.kernel(
        out_type=jax.ShapeDtypeStruct((n, d), table.dtype),
        mesh=plsc.VectorSubcoreMesh(core_axis_name="core",
                                    subcore_axis_name="subcore"),
    )
    def gather_kernel(x_hbm, i_hbm, o_hbm):
        def body(i_vmem, o_vmem):
            pltpu.sync_copy(x_hbm.at[i_vmem.at[0]], o_vmem)

        pltpu.emit_pipeline(
            body,
            grid=(n // _GATHER_WINDOW,),
            in_specs=[pl.BlockSpec((1, _GATHER_WINDOW),
                                   index_map=lambda i: (0, i))],
            out_specs=[pl.BlockSpec((_GATHER_WINDOW, d),
                                    index_map=lambda i: (i, 0))],
            core_axis_name=("core", "subcore"),
            dimension_semantics=(pltpu.PARALLEL,),
        )(i_hbm, o_hbm)

    return gather_kernel(table, idx2)


def _dense_mlp_kernel(x_ref, w1_ref, b1_ref, w2_ref, b2_ref, w3_ref, b3_ref,
                      o_ref):
    x = x_ref[...]
    h = jnp.maximum(x @ w1_ref[...] + b1_ref[...], 0.0)
    h = jnp.maximum(h @ w2_ref[...] + b2_ref[...], 0.0)
    h = jnp.maximum(h @ w3_ref[...] + b3_ref[...], 0.0)
    o_ref[...] = h


def _dense_mlp(x_pad, w1p, b1, w2, b2, w3, b3):
    bb = 2048
    full = lambda a: pl.BlockSpec(a.shape, lambda i: (0,) * a.ndim)
    return pl.pallas_call(
        _dense_mlp_kernel,
        grid=(_B // bb,),
        in_specs=[pl.BlockSpec((bb, x_pad.shape[1]), lambda i: (i, 0)),
                  full(w1p), full(b1), full(w2), full(b2), full(w3), full(b3)],
        out_specs=pl.BlockSpec((bb, _D), lambda i: (i, 0)),
        out_shape=jax.ShapeDtypeStruct((_B, _D), jnp.float32),
    )(x_pad, w1p, b1, w2, b2, w3, b3)


def _main_kernel(wide_ref, h_ref, d_ref, w1_ref, b1_ref, w2_ref, b2_ref,
                 w3_ref, b3_ref, w4_ref, b4_ref, w5_ref, b5_ref, o_ref):
    wide = wide_ref[...]
    sel = h_ref[...]
    lo = wide[:, :_D]
    hi = wide[:, _D:]
    sel2 = sel[:, None]
    w64 = jnp.where(sel2 >= 2, hi, lo)
    bits = jnp.where(jnp.bitwise_and(sel2, 1) == 1,
                     jnp.bitwise_and(w64, jnp.int32(-65536)),
                     jax.lax.shift_left(w64, 16))
    e2 = jax.lax.bitcast_convert_type(bits, jnp.float32)
    bb = e2.shape[0] // _F
    d = d_ref[...]
    c = jnp.concatenate([d[:, None, :], e2.reshape(bb, _F, _D)],
                        axis=1)
    inter = jax.lax.dot_general(
        c, c, dimension_numbers=(((2,), (2,)), ((0,), (0,))),
        preferred_element_type=jnp.float32)
    parts = [d]
    for i in range(1, _NF):
        parts.append(inter[:, i, :i])
    parts.append(jnp.zeros((bb, 1), jnp.float32))
    x = jnp.concatenate(parts, axis=1)
    x = jnp.maximum(x @ w1_ref[...] + b1_ref[...], 0.0)
    x = jnp.maximum(x @ w2_ref[...] + b2_ref[...], 0.0)
    x = jnp.maximum(x @ w3_ref[...] + b3_ref[...], 0.0)
    x = jnp.maximum(x @ w4_ref[...] + b4_ref[...], 0.0)
    o_ref[...] = x @ w5_ref[...] + b5_ref[...]


def _main(wide, hsel, d, w1p, b1, w2, b2, w3, b3, w4, b4, w5, b5):
    bb = 512
    n = wide.shape[0] // _F
    full = lambda a: pl.BlockSpec(a.shape, lambda i: (0,) * a.ndim)
    return pl.pallas_call(
        _main_kernel,
        grid=(n // bb,),
        in_specs=[pl.BlockSpec((bb * _F, 128), lambda i: (i, 0)),
                  pl.BlockSpec((bb * _F,), lambda i: (i,)),
                  pl.BlockSpec((bb, _D), lambda i: (i, 0)),
                  full(w1p), full(b1), full(w2), full(b2), full(w3), full(b3),
                  full(w4), full(b4), full(w5), full(b5)],
        out_specs=pl.BlockSpec((bb, 1), lambda i: (i, 0)),
        out_shape=jax.ShapeDtypeStruct((n, 1), jnp.float32),
    )(wide, hsel, d, w1p, b1, w2, b2, w3, b3, w4, b4, w5, b5)


def kernel(dense_features, sparse_indices, offsets, W_embed, dense_params,
           over_params):
    flat_idx = (sparse_indices + offsets[None, :]).reshape(-1).astype(jnp.int32)
    sel = flat_idx // _SPLIT4
    q = (flat_idx - sel * _SPLIT4).astype(jnp.int32)

    (w1d, b1d), (w2d, b2d), (w3d, b3d) = dense_params
    x_pad = jnp.pad(dense_features, ((0, 0), (0, 16 - dense_features.shape[1])))
    w1d_pad = jnp.pad(w1d, ((0, 16 - w1d.shape[0]), (0, 0)))

    (w1o, b1o), (w2o, b2o), (w3o, b3o), (w4o, b4o), (w5o, b5o) = over_params
    w1o_pad = jnp.pad(w1o, ((0, 416 - w1o.shape[0]), (0, 0)))

    r2 = lambda b: b.reshape(1, -1)

    wp = _pack(W_embed.T)

    half = (_B * _F) // 2
    wide1 = _sc_gather(wp, q[:half])
    wide2 = _sc_gather(wp, q[half:])

    d = _dense_mlp(x_pad, w1d_pad, r2(b1d), w2d, r2(b2d), w3d, r2(b3d))

    hsel = sel.astype(jnp.int32)
    ow = (w1o_pad, r2(b1o), w2o, r2(b2o), w3o, r2(b3o), w4o, r2(b4o), w5o,
          r2(b5o))
    logits1 = _main(wide1, hsel[:half], d[:_B // 2], *ow)
    logits2 = _main(wide2, hsel[half:], d[_B // 2:], *ow)
    return jnp.concatenate([logits1, logits2], axis=0)


# speedup vs baseline: 1.1872x; 1.0011x over previous
import numpy as np
import jax
import jax.numpy as jnp
from jax.experimental import pallas as pl
from jax.experimental.pallas import tpu as pltpu
from jax.experimental.pallas import tpu_sc as plsc

_B = 16384
_F = 26
_D = 64
_NF = _F + 1
_V = 2600000

_SPLIT4 = 655360
_PACK_C = 8192
_N_IN_BLOCKS = (_V + _PACK_C - 1) // _PACK_C - 1

_GATHER_WINDOW = 128


def _pack_kernel(x0_ref, x1_ref, x2_ref, x3_ref, o_ref):
    t = jnp.transpose(jnp.concatenate([x0_ref[...], x2_ref[...]],
                                      axis=0))
    u = jnp.transpose(jnp.concatenate([x1_ref[...], x3_ref[...]],
                                      axis=0))
    tb = jax.lax.bitcast_convert_type(t, jnp.int32) + jnp.int32(0x8000)
    ub = jax.lax.bitcast_convert_type(u, jnp.int32) + jnp.int32(0x8000)
    o_ref[...] = jnp.bitwise_or(
        jax.lax.shift_right_logical(tb, 16),
        jnp.bitwise_and(ub, jnp.int32(-65536)))


def _pack(wt):
    nj = _SPLIT4 // _PACK_C

    def imap(k):
        return lambda j: (0, jnp.minimum(j + k * nj, _N_IN_BLOCKS))

    return pl.pallas_call(
        _pack_kernel,
        grid=(nj,),
        in_specs=[pl.BlockSpec((_D, _PACK_C), imap(0)),
                  pl.BlockSpec((_D, _PACK_C), imap(1)),
                  pl.BlockSpec((_D, _PACK_C), imap(2)),
                  pl.BlockSpec((_D, _PACK_C), imap(3))],
        out_specs=pl.BlockSpec((_PACK_C, 128), lambda j: (j, 0)),
        out_shape=jax.ShapeDtypeStruct((_SPLIT4, 128), jnp.int32),
    )(wt, wt, wt, wt)


def _sc_gather(table, flat_idx):
    n = flat_idx.shape[0]
    d = table.shape[1]
    idx2 = flat_idx.reshape(1, n)

    @pl.kernel(
        out_type=jax.ShapeDtypeStruct((n, d), table.dtype),
        mesh=plsc.VectorSubcoreMesh(core_axis_name="core",
                                    subcore_axis_name="subcore"),
    )
    def gather_kernel(x_hbm, i_hbm, o_hbm):
        def body(i_vmem, o_vmem):
            pltpu.sync_copy(x_hbm.at[i_vmem.at[0]], o_vmem)

        pltpu.emit_pipeline(
            body,
            grid=(n // _GATHER_WINDOW,),
            in_specs=[pl.BlockSpec((1, _GATHER_WINDOW),
                                   index_map=lambda i: (0, i))],
            out_specs=[pl.BlockSpec((_GATHER_WINDOW, d),
                                    index_map=lambda i: (i, 0))],
            core_axis_name=("core", "subcore"),
            dimension_semantics=(pltpu.PARALLEL,),
        )(i_hbm, o_hbm)

    return gather_kernel(table, idx2)


def _dense_mlp_kernel(x_ref, w1_ref, b1_ref, w2_ref, b2_ref, w3_ref, b3_ref,
                      o_ref):
    x = x_ref[...]
    h = jnp.maximum(x @ w1_ref[...] + b1_ref[...], 0.0)
    h = jnp.maximum(h @ w2_ref[...] + b2_ref[...], 0.0)
    h = jnp.maximum(h @ w3_ref[...] + b3_ref[...], 0.0)
    o_ref[...] = h


def _dense_mlp(x_pad, w1p, b1, w2, b2, w3, b3):
    bb = 2048
    full = lambda a: pl.BlockSpec(a.shape, lambda i: (0,) * a.ndim)
    return pl.pallas_call(
        _dense_mlp_kernel,
        grid=(_B // bb,),
        in_specs=[pl.BlockSpec((bb, x_pad.shape[1]), lambda i: (i, 0)),
                  full(w1p), full(b1), full(w2), full(b2), full(w3), full(b3)],
        out_specs=pl.BlockSpec((bb, _D), lambda i: (i, 0)),
        out_shape=jax.ShapeDtypeStruct((_B, _D), jnp.float32),
    )(x_pad, w1p, b1, w2, b2, w3, b3)


def _main_kernel(wide_ref, h_ref, d_ref, w1_ref, b1_ref, w2_ref, b2_ref,
                 w3_ref, b3_ref, w4_ref, b4_ref, w5_ref, b5_ref, o_ref):
    wide = wide_ref[...]
    sel = h_ref[...]
    lo = wide[:, :_D]
    hi = wide[:, _D:]
    sel2 = sel[:, None]
    w64 = jnp.where(sel2 >= 2, hi, lo)
    bits = jnp.where(jnp.bitwise_and(sel2, 1) == 1,
                     jnp.bitwise_and(w64, jnp.int32(-65536)),
                     jax.lax.shift_left(w64, 16))
    e2 = jax.lax.bitcast_convert_type(bits, jnp.float32)
    bb = e2.shape[0] // _F
    d = d_ref[...]
    c = jnp.concatenate([d[:, None, :], e2.reshape(bb, _F, _D)],
                        axis=1)
    inter = jax.lax.dot_general(
        c, c, dimension_numbers=(((2,), (2,)), ((0,), (0,))),
        preferred_element_type=jnp.float32)
    parts = [d]
    for i in range(1, _NF):
        parts.append(inter[:, i, :i])
    parts.append(jnp.zeros((bb, 1), jnp.float32))
    x = jnp.concatenate(parts, axis=1)
    x = jnp.maximum(x @ w1_ref[...] + b1_ref[...], 0.0)
    x = jnp.maximum(x @ w2_ref[...] + b2_ref[...], 0.0)
    x = jnp.maximum(x @ w3_ref[...] + b3_ref[...], 0.0)
    x = jnp.maximum(x @ w4_ref[...] + b4_ref[...], 0.0)
    o_ref[...] = x @ w5_ref[...] + b5_ref[...]


def _main(wide, hsel, d, w1p, b1, w2, b2, w3, b3, w4, b4, w5, b5):
    bb = 512
    n = wide.shape[0] // _F
    full = lambda a: pl.BlockSpec(a.shape, lambda i: (0,) * a.ndim)
    return pl.pallas_call(
        _main_kernel,
        grid=(n // bb,),
        in_specs=[pl.BlockSpec((bb * _F, 128), lambda i: (i, 0)),
                  pl.BlockSpec((bb * _F,), lambda i: (i,)),
                  pl.BlockSpec((bb, _D), lambda i: (i, 0)),
                  full(w1p), full(b1), full(w2), full(b2), full(w3), full(b3),
                  full(w4), full(b4), full(w5), full(b5)],
        out_specs=pl.BlockSpec((bb, 1), lambda i: (i, 0)),
        out_shape=jax.ShapeDtypeStruct((n, 1), jnp.float32),
    )(wide, hsel, d, w1p, b1, w2, b2, w3, b3, w4, b4, w5, b5)


def kernel(dense_features, sparse_indices, offsets, W_embed, dense_params,
           over_params):
    flat_idx = (sparse_indices + offsets[None, :]).reshape(-1).astype(jnp.int32)
    sel = flat_idx // _SPLIT4
    q = (flat_idx - sel * _SPLIT4).astype(jnp.int32)

    (w1d, b1d), (w2d, b2d), (w3d, b3d) = dense_params
    x_pad = jnp.pad(dense_features, ((0, 0), (0, 16 - dense_features.shape[1])))
    w1d_pad = jnp.pad(w1d, ((0, 16 - w1d.shape[0]), (0, 0)))

    (w1o, b1o), (w2o, b2o), (w3o, b3o), (w4o, b4o), (w5o, b5o) = over_params
    w1o_pad = jnp.pad(w1o, ((0, 416 - w1o.shape[0]), (0, 0)))

    r2 = lambda b: b.reshape(1, -1)

    wp = _pack(W_embed.T)

    half = (_B * _F) // 2
    wide1 = _sc_gather(wp, q[:half])
    wide2 = _sc_gather(wp, q[half:])

    d = _dense_mlp(x_pad, w1d_pad, r2(b1d), w2d, r2(b2d), w3d, r2(b3d))

    hsel = sel.astype(jnp.int32)
    ow = (w1o_pad, r2(b1o), w2o, r2(b2o), w3o, r2(b3o), w4o, r2(b4o), w5o,
          r2(b5o))
    logits1 = _main(wide1, hsel[:half], d[:_B // 2], *ow)
    logits2 = _main(wide2, hsel[half:], d[_B // 2:], *ow)
    return jnp.concatenate([logits1, logits2], axis=0)


# speedup vs baseline: 1.2188x; 1.0266x over previous
import numpy as np
import jax
import jax.numpy as jnp
from jax.experimental import pallas as pl
from jax.experimental.pallas import tpu as pltpu
from jax.experimental.pallas import tpu_sc as plsc

_B = 16384
_F = 26
_D = 64
_NF = _F + 1
_V = 2600000

_SPLIT4 = 655360
_PACK_C = 8192
_N_IN_BLOCKS = (_V + _PACK_C - 1) // _PACK_C - 1

_GATHER_WINDOW = 128


def _pack_kernel(x0_ref, x1_ref, x2_ref, x3_ref, o_ref):
    t = jnp.transpose(jnp.concatenate([x0_ref[...], x2_ref[...]],
                                      axis=0))
    u = jnp.transpose(jnp.concatenate([x1_ref[...], x3_ref[...]],
                                      axis=0))
    tb = jax.lax.bitcast_convert_type(t, jnp.int32) + jnp.int32(0x8000)
    ub = jax.lax.bitcast_convert_type(u, jnp.int32) + jnp.int32(0x8000)
    o_ref[...] = jnp.bitwise_or(
        jax.lax.shift_right_logical(tb, 16),
        jnp.bitwise_and(ub, jnp.int32(-65536)))


def _pack(wt):
    nj = _SPLIT4 // _PACK_C

    def imap(k):
        return lambda j: (0, jnp.minimum(j + k * nj, _N_IN_BLOCKS))

    return pl.pallas_call(
        _pack_kernel,
        grid=(nj,),
        in_specs=[pl.BlockSpec((_D, _PACK_C), imap(0)),
                  pl.BlockSpec((_D, _PACK_C), imap(1)),
                  pl.BlockSpec((_D, _PACK_C), imap(2)),
                  pl.BlockSpec((_D, _PACK_C), imap(3))],
        out_specs=pl.BlockSpec((_PACK_C, 128), lambda j: (j, 0)),
        out_shape=jax.ShapeDtypeStruct((_SPLIT4, 128), jnp.int32),
    )(wt, wt, wt, wt)


def _sc_gather(table, flat_idx):
    n = flat_idx.shape[0]
    d = table.shape[1]
    idx2 = flat_idx.reshape(1, n)

    @pl.kernel(
        out_type=jax.ShapeDtypeStruct((n, d), table.dtype),
        mesh=plsc.VectorSubcoreMesh(core_axis_name="core",
                                    subcore_axis_name="subcore"),
    )
    def gather_kernel(x_hbm, i_hbm, o_hbm):
        def body(i_vmem, o_vmem):
            pltpu.sync_copy(x_hbm.at[i_vmem.at[0]], o_vmem)

        pltpu.emit_pipeline(
            body,
            grid=(n // _GATHER_WINDOW,),
            in_specs=[pl.BlockSpec((1, _GATHER_WINDOW),
                                   index_map=lambda i: (0, i))],
            out_specs=[pl.BlockSpec((_GATHER_WINDOW, d),
                                    index_map=lambda i: (i, 0))],
            core_axis_name=("core", "subcore"),
            dimension_semantics=(pltpu.PARALLEL,),
        )(i_hbm, o_hbm)

    return gather_kernel(table, idx2)


def _dense_mlp_kernel(x_ref, w1_ref, b1_ref, w2_ref, b2_ref, w3_ref, b3_ref,
                      o_ref):
    x = x_ref[...]
    h = jnp.maximum(x @ w1_ref[...] + b1_ref[...], 0.0)
    h = jnp.maximum(h @ w2_ref[...] + b2_ref[...], 0.0)
    h = jnp.maximum(h @ w3_ref[...] + b3_ref[...], 0.0)
    o_ref[...] = h


def _dense_mlp(x_pad, w1p, b1, w2, b2, w3, b3):
    bb = 2048
    full = lambda a: pl.BlockSpec(a.shape, lambda i: (0,) * a.ndim)
    return pl.pallas_call(
        _dense_mlp_kernel,
        grid=(_B // bb,),
        in_specs=[pl.BlockSpec((bb, x_pad.shape[1]), lambda i: (i, 0)),
                  full(w1p), full(b1), full(w2), full(b2), full(w3), full(b3)],
        out_specs=pl.BlockSpec((bb, _D), lambda i: (i, 0)),
        out_shape=jax.ShapeDtypeStruct((_B, _D), jnp.float32),
    )(x_pad, w1p, b1, w2, b2, w3, b3)


def _main_kernel(wide_ref, h_ref, d_ref, w1_ref, b1_ref, w2_ref, b2_ref,
                 w3_ref, b3_ref, w4_ref, b4_ref, w5_ref, b5_ref, o_ref):
    wide = wide_ref[...]
    sel = h_ref[...]
    lo = wide[:, :_D]
    hi = wide[:, _D:]
    sel2 = sel[:, None]
    w64 = jnp.where(sel2 >= 2, hi, lo)
    bits = jnp.where(jnp.bitwise_and(sel2, 1) == 1,
                     jnp.bitwise_and(w64, jnp.int32(-65536)),
                     jax.lax.shift_left(w64, 16))
    e2 = jax.lax.bitcast_convert_type(bits, jnp.float32)
    bb = e2.shape[0] // _F
    d = d_ref[...]
    c = jnp.concatenate([d[:, None, :], e2.reshape(bb, _F, _D)],
                        axis=1)
    inter = jax.lax.dot_general(
        c, c, dimension_numbers=(((2,), (2,)), ((0,), (0,))),
        preferred_element_type=jnp.float32)
    parts = [d]
    for i in range(1, _NF):
        parts.append(inter[:, i, :i])
    parts.append(jnp.zeros((bb, 1), jnp.float32))
    x = jnp.concatenate(parts, axis=1)
    x = jnp.maximum(x @ w1_ref[...] + b1_ref[...], 0.0)
    x = jnp.maximum(x @ w2_ref[...] + b2_ref[...], 0.0)
    x = jnp.maximum(x @ w3_ref[...] + b3_ref[...], 0.0)
    x = jnp.maximum(x @ w4_ref[...] + b4_ref[...], 0.0)
    o_ref[...] = x @ w5_ref[...] + b5_ref[...]


def _main(wide, hsel, d, w1p, b1, w2, b2, w3, b3, w4, b4, w5, b5):
    bb = 512
    n = wide.shape[0] // _F
    full = lambda a: pl.BlockSpec(a.shape, lambda i: (0,) * a.ndim)
    return pl.pallas_call(
        _main_kernel,
        grid=(n // bb,),
        in_specs=[pl.BlockSpec((bb * _F, 128), lambda i: (i, 0)),
                  pl.BlockSpec((bb * _F,), lambda i: (i,)),
                  pl.BlockSpec((bb, _D), lambda i: (i, 0)),
                  full(w1p), full(b1), full(w2), full(b2), full(w3), full(b3),
                  full(w4), full(b4), full(w5), full(b5)],
        out_specs=pl.BlockSpec((bb, 1), lambda i: (i, 0)),
        out_shape=jax.ShapeDtypeStruct((n, 1), jnp.float32),
    )(wide, hsel, d, w1p, b1, w2, b2, w3, b3, w4, b4, w5, b5)


def kernel(dense_features, sparse_indices, offsets, W_embed, dense_params,
           over_params):
    flat_idx = (sparse_indices + offsets[None, :]).reshape(-1).astype(jnp.int32)
    sel = flat_idx // _SPLIT4
    q = (flat_idx - sel * _SPLIT4).astype(jnp.int32)

    (w1d, b1d), (w2d, b2d), (w3d, b3d) = dense_params
    x_pad = jnp.pad(dense_features, ((0, 0), (0, 16 - dense_features.shape[1])))
    w1d_pad = jnp.pad(w1d, ((0, 16 - w1d.shape[0]), (0, 0)))

    (w1o, b1o), (w2o, b2o), (w3o, b3o), (w4o, b4o), (w5o, b5o) = over_params
    w1o_pad = jnp.pad(w1o, ((0, 416 - w1o.shape[0]), (0, 0)))

    r2 = lambda b: b.reshape(1, -1)

    wp = _pack(W_embed.T)

    nq = 4
    qn = (_B * _F) // nq
    bn = _B // nq
    wides = [_sc_gather(wp, q[i * qn:(i + 1) * qn]) for i in range(nq)]

    d = _dense_mlp(x_pad, w1d_pad, r2(b1d), w2d, r2(b2d), w3d, r2(b3d))

    hsel = sel.astype(jnp.int32)
    ow = (w1o_pad, r2(b1o), w2o, r2(b2o), w3o, r2(b3o), w4o, r2(b4o), w5o,
          r2(b5o))
    logits = [
        _main(wides[i], hsel[i * qn:(i + 1) * qn], d[i * bn:(i + 1) * bn], *ow)
        for i in range(nq)
    ]
    return jnp.concatenate(logits, axis=0)
